# baseline (speedup 1.0000x reference)
import functools

import jax
import jax.numpy as jnp
from jax import lax
from jax.experimental import pallas as pl
from jax.experimental.pallas import tpu as pltpu
from jax.experimental.pallas import tpu_sc as plsc

B, L, V, D, H, C = 4096, 50, 1000000, 64, 256, 10

NC, NS = 2, 16
NW = NC * NS
BAGS_PER_W = B // NW
BAGS_PER_STEP = 2
IDX_PER_STEP = BAGS_PER_STEP * L
STEPS = BAGS_PER_W // BAGS_PER_STEP
ROWS_PER_W = STEPS


def _sc_embed_bag_mean(x2d, table):
    mesh = plsc.VectorSubcoreMesh(core_axis_name="c", subcore_axis_name="s")

    @functools.partial(
        pl.kernel,
        out_type=jax.ShapeDtypeStruct((B, D), jnp.float32),
        mesh=mesh,
        scratch_types=[
            pltpu.VMEM((ROWS_PER_W, IDX_PER_STEP), jnp.int32),
            pltpu.VMEM((IDX_PER_STEP, D), jnp.float32),
            pltpu.VMEM((BAGS_PER_W, D), jnp.float32),
            pltpu.SemaphoreType.DMA,
        ],
        compiler_params=pltpu.CompilerParams(use_tc_tiling_on_sc=False),
    )
    def k(table_hbm, x_hbm, out_hbm, idx_v, rows_v, acc_v, sem):
        wid = lax.axis_index("s") * NC + lax.axis_index("c")
        pltpu.sync_copy(x_hbm.at[pl.ds(wid * ROWS_PER_W, ROWS_PER_W)], idx_v)

        def step(g, carry):
            pltpu.async_copy(table_hbm.at[idx_v.at[g]], rows_v, sem).wait()
            for b in range(BAGS_PER_STEP):
                for d in range(D // 16):
                    def body(l, a):
                        return a + rows_v[b * L + l, pl.ds(d * 16, 16)]
                    acc = lax.fori_loop(
                        0, L, body, jnp.zeros((16,), jnp.float32))
                    acc_v[g * BAGS_PER_STEP + b, pl.ds(d * 16, 16)] = (
                        acc * (1.0 / L))
            return carry

        lax.fori_loop(0, STEPS, step, 0)
        pltpu.sync_copy(acc_v, out_hbm.at[pl.ds(wid * BAGS_PER_W, BAGS_PER_W)])

    return k(table, x2d)


_MLP_BLK = 512


def _mlp_body(bag_ref, w1_ref, b1_ref, w2_ref, b2_ref, out_ref):
    h = lax.dot_general(bag_ref[...], w1_ref[...],
                        (((1,), (1,)), ((), ())),
                        preferred_element_type=jnp.float32) + b1_ref[...]
    o = lax.dot_general(h, w2_ref[...],
                        (((1,), (1,)), ((), ())),
                        preferred_element_type=jnp.float32) + b2_ref[...]
    m = jnp.max(o, axis=-1, keepdims=True)
    e = jnp.exp(o - m)
    out_ref[...] = e / jnp.sum(e, axis=-1, keepdims=True)


def _tc_mlp_softmax(bag, W1, b1, W2, b2):
    return pl.pallas_call(
        _mlp_body,
        out_shape=jax.ShapeDtypeStruct((B, C), jnp.float32),
        grid=(B // _MLP_BLK,),
        in_specs=[
            pl.BlockSpec((_MLP_BLK, D), lambda i: (i, 0)),
            pl.BlockSpec((H, D), lambda i: (0, 0)),
            pl.BlockSpec((1, H), lambda i: (0, 0)),
            pl.BlockSpec((C, H), lambda i: (0, 0)),
            pl.BlockSpec((1, C), lambda i: (0, 0)),
        ],
        out_specs=pl.BlockSpec((_MLP_BLK, C), lambda i: (i, 0)),
    )(bag, W1, b1, W2, b2)


def kernel(x, table, W1, b1, W2, b2):
    x2d = x.reshape(-1, IDX_PER_STEP).astype(jnp.int32)
    bag = _sc_embed_bag_mean(x2d, table)
    return _tc_mlp_softmax(bag, W1, b1.reshape(1, H), W2, b2.reshape(1, C))


# speedup vs baseline: 1.1699x; 1.1699x over previous
import functools

import jax
import jax.numpy as jnp
from jax import lax
from jax.experimental import pallas as pl
from jax.experimental.pallas import tpu as pltpu
from jax.experimental.pallas import tpu_sc as plsc

B, L, V, D, H, C = 4096, 50, 1000000, 64, 256, 10

NC, NS = 2, 16
NW = NC * NS
BAGS_PER_W = B // NW
BAGS_PER_STEP = 2
IDX_PER_STEP = BAGS_PER_STEP * L
STEPS = BAGS_PER_W // BAGS_PER_STEP
ROWS_PER_W = STEPS


def _sc_embed_bag_mean(x2d, table):
    mesh = plsc.VectorSubcoreMesh(core_axis_name="c", subcore_axis_name="s")

    @functools.partial(
        pl.kernel,
        out_type=jax.ShapeDtypeStruct((B, D), jnp.float32),
        mesh=mesh,
        scratch_types=[
            pltpu.VMEM((ROWS_PER_W, IDX_PER_STEP), jnp.int32),
            pltpu.VMEM((IDX_PER_STEP, D), jnp.float32),
            pltpu.VMEM((IDX_PER_STEP, D), jnp.float32),
            pltpu.VMEM((BAGS_PER_W, D), jnp.float32),
            pltpu.SemaphoreType.DMA,
            pltpu.SemaphoreType.DMA,
        ],
        compiler_params=pltpu.CompilerParams(use_tc_tiling_on_sc=False),
    )
    def k(table_hbm, x_hbm, out_hbm, idx_v, rows_a, rows_b, acc_v,
          sem_a, sem_b):
        wid = lax.axis_index("s") * NC + lax.axis_index("c")
        pltpu.sync_copy(x_hbm.at[pl.ds(wid * ROWS_PER_W, ROWS_PER_W)], idx_v)

        def fire(g, buf, sem):
            pltpu.async_copy(table_hbm.at[idx_v.at[g]], buf, sem)

        def wait(buf, sem):
            pltpu.make_async_copy(
                table_hbm.at[pl.ds(0, IDX_PER_STEP)], buf, sem).wait()

        def accum(g, buf):
            for b in range(BAGS_PER_STEP):
                accs = [jnp.zeros((16,), jnp.float32)] * (D // 16)
                for l in range(L):
                    for d in range(D // 16):
                        accs[d] = accs[d] + buf[b * L + l, pl.ds(d * 16, 16)]
                for d in range(D // 16):
                    acc_v[g * BAGS_PER_STEP + b, pl.ds(d * 16, 16)] = (
                        accs[d] * (1.0 / L))

        fire(0, rows_a, sem_a)

        def step2(g2, carry):
            g = g2 * 2
            fire(g + 1, rows_b, sem_b)
            wait(rows_a, sem_a)
            accum(g, rows_a)

            @pl.when(g2 < STEPS // 2 - 1)
            def _():
                fire(g + 2, rows_a, sem_a)

            wait(rows_b, sem_b)
            accum(g + 1, rows_b)
            return carry

        lax.fori_loop(0, STEPS // 2, step2, 0)
        pltpu.sync_copy(acc_v, out_hbm.at[pl.ds(wid * BAGS_PER_W, BAGS_PER_W)])

    return k(table, x2d)


_MLP_BLK = 512


def _mlp_body(bag_ref, w1_ref, b1_ref, w2_ref, b2_ref, out_ref):
    h = lax.dot_general(bag_ref[...], w1_ref[...],
                        (((1,), (1,)), ((), ())),
                        preferred_element_type=jnp.float32) + b1_ref[...]
    o = lax.dot_general(h, w2_ref[...],
                        (((1,), (1,)), ((), ())),
                        preferred_element_type=jnp.float32) + b2_ref[...]
    m = jnp.max(o, axis=-1, keepdims=True)
    e = jnp.exp(o - m)
    out_ref[...] = e / jnp.sum(e, axis=-1, keepdims=True)


def _tc_mlp_softmax(bag, W1, b1, W2, b2):
    return pl.pallas_call(
        _mlp_body,
        out_shape=jax.ShapeDtypeStruct((B, C), jnp.float32),
        grid=(B // _MLP_BLK,),
        in_specs=[
            pl.BlockSpec((_MLP_BLK, D), lambda i: (i, 0)),
            pl.BlockSpec((H, D), lambda i: (0, 0)),
            pl.BlockSpec((1, H), lambda i: (0, 0)),
            pl.BlockSpec((C, H), lambda i: (0, 0)),
            pl.BlockSpec((1, C), lambda i: (0, 0)),
        ],
        out_specs=pl.BlockSpec((_MLP_BLK, C), lambda i: (i, 0)),
    )(bag, W1, b1, W2, b2)


def kernel(x, table, W1, b1, W2, b2):
    x2d = x.reshape(-1, IDX_PER_STEP).astype(jnp.int32)
    bag = _sc_embed_bag_mean(x2d, table)
    return _tc_mlp_softmax(bag, W1, b1.reshape(1, H), W2, b2.reshape(1, C))
